# Initial kernel scaffold; baseline (speedup 1.0000x reference)
import jax
import jax.numpy as jnp
from jax.experimental import pallas as pl


def kernel(features, edge_index, positive_edge_pairs, negative_edge_pairs, W1, b1, W2, b2):
    raise NotImplementedError("write your pallas kernel here")


# speedup vs baseline: 5.1309x; 5.1309x over previous
import functools
import jax
import jax.numpy as jnp
from jax import lax
from jax.experimental import pallas as pl
from jax.experimental.pallas import tpu as pltpu
from jax.experimental.pallas import tpu_sc as plsc

NC = 2
NS = 16
L = 16
NW = NC * NS

N_NODES = 10000
N_PAD = 10240
EDGE_BLK = 128


def _mm_bias_body(x_ref, w_ref, b_ref, o_ref):
    o_ref[...] = (
        jnp.dot(x_ref[...], w_ref[...], preferred_element_type=jnp.float32)
        + b_ref[...]
    )


def _tc_matmul_bias(x, w, b, block_rows=1000):
    n, d = x.shape
    h = w.shape[1]
    b2d = b.reshape(1, h)
    return pl.pallas_call(
        _mm_bias_body,
        grid=(n // block_rows,),
        in_specs=[
            pl.BlockSpec((block_rows, d), lambda i: (i, 0)),
            pl.BlockSpec((d, h), lambda i: (0, 0)),
            pl.BlockSpec((1, h), lambda i: (0, 0)),
        ],
        out_specs=pl.BlockSpec((block_rows, h), lambda i: (i, 0)),
        out_shape=jax.ShapeDtypeStruct((n, h), jnp.float32),
    )(x, w, b2d)


def _fuse_body(p0_ref, p1_ref, w_ref, b_ref, o_ref):
    h = jnp.maximum(p0_ref[...] + p1_ref[...], 0.0)
    o_ref[...] = (
        jnp.dot(h, w_ref[...], preferred_element_type=jnp.float32) + b_ref[...]
    )


def _tc_relu_combine_matmul(p0, p1, w, b, block_rows=1000):
    n, d = p0.shape
    h = w.shape[1]
    b2d = b.reshape(1, h)
    return pl.pallas_call(
        _fuse_body,
        grid=(n // block_rows,),
        in_specs=[
            pl.BlockSpec((block_rows, d), lambda i: (i, 0)),
            pl.BlockSpec((block_rows, d), lambda i: (i, 0)),
            pl.BlockSpec((d, h), lambda i: (0, 0)),
            pl.BlockSpec((1, h), lambda i: (0, 0)),
        ],
        out_specs=pl.BlockSpec((block_rows, h), lambda i: (i, 0)),
        out_shape=jax.ShapeDtypeStruct((n, h), jnp.float32),
    )(p0, p1, w, b2d)


def _add_body(p0_ref, p1_ref, o_ref):
    o_ref[...] = p0_ref[...] + p1_ref[...]


def _tc_add(p0, p1, block_rows=2000):
    n, d = p0.shape
    return pl.pallas_call(
        _add_body,
        grid=(n // block_rows,),
        in_specs=[
            pl.BlockSpec((block_rows, d), lambda i: (i, 0)),
            pl.BlockSpec((block_rows, d), lambda i: (i, 0)),
        ],
        out_specs=pl.BlockSpec((block_rows, d), lambda i: (i, 0)),
        out_shape=jax.ShapeDtypeStruct((n, d), jnp.float32),
    )(p0, p1)


def _segsum_body(n_nodes, d, n_blk, hw_hbm, src_hbm, dst_hbm, out_hbm,
                 sidx, didx, rows, agg_sh, sem):
    cid = lax.axis_index("c")
    sid = lax.axis_index("s")
    wid = sid * NC + cid

    d_grp = d // L
    stripe = n_nodes // NS

    def zero_row(k, _):
        i = k // d_grp
        j = k % d_grp
        rows[i, pl.ds(j * L, L)] = jnp.zeros((L,), jnp.float32)
        return 0

    lax.fori_loop(0, EDGE_BLK * d_grp, zero_row, 0)
    base = sid * stripe
    for k in range(stripe // EDGE_BLK):
        pltpu.sync_copy(rows, agg_sh.at[pl.ds(base + k * EDGE_BLK, EDGE_BLK)])
    plsc.subcore_barrier()

    nb_w = (n_blk - wid + NW - 1) // NW

    def edge_step(i, _):
        off = (wid + i * NW) * EDGE_BLK
        pltpu.sync_copy(src_hbm.at[pl.ds(off, EDGE_BLK)], sidx.at[0])
        pltpu.sync_copy(dst_hbm.at[pl.ds(off, EDGE_BLK)], didx.at[0])
        pltpu.async_copy(hw_hbm.at[sidx.at[0]], rows, sem).wait()
        pltpu.sync_copy(rows, agg_sh.at[didx.at[0]], add=True)
        return 0

    lax.fori_loop(0, nb_w, edge_step, 0)
    plsc.subcore_barrier()

    pltpu.sync_copy(agg_sh.at[pl.ds(base, stripe)],
                    out_hbm.at[cid, pl.ds(base, stripe)])


def _sc_segsum(hw, src, dst, n_nodes):
    e = src.shape[0]
    d = hw.shape[1]
    n_blk = e // EDGE_BLK
    mesh = plsc.VectorSubcoreMesh(core_axis_name="c", subcore_axis_name="s")
    kern = pl.kernel(
        functools.partial(_segsum_body, n_nodes, d, n_blk),
        out_type=jax.ShapeDtypeStruct((NC, n_nodes, d), jnp.float32),
        mesh=mesh,
        scratch_types=[
            pltpu.VMEM((1, EDGE_BLK), jnp.int32),
            pltpu.VMEM((1, EDGE_BLK), jnp.int32),
            pltpu.VMEM((EDGE_BLK, d), jnp.float32),
            pltpu.VMEM_SHARED((n_nodes, d), jnp.float32),
            pltpu.SemaphoreType.DMA,
        ],
        compiler_params=pltpu.CompilerParams(use_tc_tiling_on_sc=False),
    )
    return kern(hw, src, dst)


def _decoder_body(d, n_blk, h2_hbm, aidx_hbm, bidx_hbm, out_hbm,
                  aidx, bidx, urows, vrows, outv, sem):
    cid = lax.axis_index("c")
    sid = lax.axis_index("s")
    wid = sid * NC + cid
    blk_per_w = n_blk // NW

    def block_step(i, _):
        off = (wid * blk_per_w + i) * EDGE_BLK
        pltpu.sync_copy(aidx_hbm.at[pl.ds(off, EDGE_BLK)], aidx.at[0])
        pltpu.sync_copy(bidx_hbm.at[pl.ds(off, EDGE_BLK)], bidx.at[0])
        pltpu.async_copy(h2_hbm.at[aidx.at[0]], urows, sem).wait()
        pltpu.async_copy(h2_hbm.at[bidx.at[0]], vrows, sem).wait()
        for g in range(EDGE_BLK // L):
            pv = g * L + lax.iota(jnp.int32, L)

            def dot_step(dd, acc):
                dv = jnp.full((L,), dd, jnp.int32)
                ua = plsc.load_gather(urows, [pv, dv])
                vb = plsc.load_gather(vrows, [pv, dv])
                return acc + ua * vb

            acc = lax.fori_loop(0, d, dot_step, jnp.zeros((L,), jnp.float32))
            outv[0, pl.ds(g * L, L)] = acc
        pltpu.sync_copy(outv.at[0], out_hbm.at[pl.ds(off, EDGE_BLK)])
        return 0

    lax.fori_loop(0, blk_per_w, block_step, 0)


def _sc_decoder(h2, aidx, bidx):
    p = aidx.shape[0]
    d = h2.shape[1]
    n_blk = p // EDGE_BLK
    mesh = plsc.VectorSubcoreMesh(core_axis_name="c", subcore_axis_name="s")
    kern = pl.kernel(
        functools.partial(_decoder_body, d, n_blk),
        out_type=jax.ShapeDtypeStruct((p,), jnp.float32),
        mesh=mesh,
        scratch_types=[
            pltpu.VMEM((1, EDGE_BLK), jnp.int32),
            pltpu.VMEM((1, EDGE_BLK), jnp.int32),
            pltpu.VMEM((EDGE_BLK, d), jnp.float32),
            pltpu.VMEM((EDGE_BLK, d), jnp.float32),
            pltpu.VMEM((1, EDGE_BLK), jnp.float32),
            pltpu.SemaphoreType.DMA,
        ],
        compiler_params=pltpu.CompilerParams(
            use_tc_tiling_on_sc=False, needs_layout_passes=False),
    )
    return kern(h2, aidx, bidx)


def kernel(features, edge_index, positive_edge_pairs, negative_edge_pairs,
           W1, b1, W2, b2):
    src = edge_index[0].astype(jnp.int32)
    dst = edge_index[1].astype(jnp.int32)

    hw1 = _tc_matmul_bias(features, W1, b1)
    part1 = _sc_segsum(hw1, src, dst, N_PAD)

    hw2 = _tc_relu_combine_matmul(part1[0], part1[1], W2, b2, block_rows=1024)
    part2 = _sc_segsum(hw2, src, dst, N_PAD)
    h2 = _tc_add(part2[0], part2[1], block_rows=2048)

    all_pairs = jnp.concatenate(
        (positive_edge_pairs, negative_edge_pairs), axis=-1).astype(jnp.int32)
    npairs = all_pairs.shape[1]
    pad = (-npairs) % (NW * EDGE_BLK)
    aidx = jnp.pad(all_pairs[0], (0, pad))
    bidx = jnp.pad(all_pairs[1], (0, pad))
    out = _sc_decoder(h2, aidx, bidx)
    return out[:npairs]


# speedup vs baseline: 8.8227x; 1.7195x over previous
import functools
import jax
import jax.numpy as jnp
from jax import lax
from jax.experimental import pallas as pl
from jax.experimental.pallas import tpu as pltpu
from jax.experimental.pallas import tpu_sc as plsc

NC = 2
NS = 16
L = 16
NW = NC * NS

N_NODES = 10000
N_PAD = 10240
EBLK = 100
EPW = 10000
NBLK_W = EPW // EBLK
PBLK = 128


def _mm_bias_body(x_ref, w_ref, b_ref, o_ref):
    o_ref[...] = (
        jnp.dot(x_ref[...], w_ref[...], preferred_element_type=jnp.float32)
        + b_ref[...]
    )


def _tc_matmul_bias(x, w, b, block_rows=1000):
    n, d = x.shape
    h = w.shape[1]
    b2d = b.reshape(1, h)
    return pl.pallas_call(
        _mm_bias_body,
        grid=(n // block_rows,),
        in_specs=[
            pl.BlockSpec((block_rows, d), lambda i: (i, 0)),
            pl.BlockSpec((d, h), lambda i: (0, 0)),
            pl.BlockSpec((1, h), lambda i: (0, 0)),
        ],
        out_specs=pl.BlockSpec((block_rows, h), lambda i: (i, 0)),
        out_shape=jax.ShapeDtypeStruct((n, h), jnp.float32),
    )(x, w, b2d)


def _fuse_body(p0_ref, p1_ref, w_ref, b_ref, o_ref):
    h = jnp.maximum(p0_ref[...] + p1_ref[...], 0.0)
    o_ref[...] = (
        jnp.dot(h, w_ref[...], preferred_element_type=jnp.float32) + b_ref[...]
    )


def _tc_relu_combine_matmul(p0, p1, w, b, block_rows=1024):
    n, d = p0.shape
    h = w.shape[1]
    b2d = b.reshape(1, h)
    return pl.pallas_call(
        _fuse_body,
        grid=(n // block_rows,),
        in_specs=[
            pl.BlockSpec((block_rows, d), lambda i: (i, 0)),
            pl.BlockSpec((block_rows, d), lambda i: (i, 0)),
            pl.BlockSpec((d, h), lambda i: (0, 0)),
            pl.BlockSpec((1, h), lambda i: (0, 0)),
        ],
        out_specs=pl.BlockSpec((block_rows, h), lambda i: (i, 0)),
        out_shape=jax.ShapeDtypeStruct((n, h), jnp.float32),
    )(p0, p1, w, b2d)


def _add_body(p0_ref, p1_ref, o_ref):
    o_ref[...] = p0_ref[...] + p1_ref[...]


def _tc_add(p0, p1, block_rows=2048):
    n, d = p0.shape
    return pl.pallas_call(
        _add_body,
        grid=(n // block_rows,),
        in_specs=[
            pl.BlockSpec((block_rows, d), lambda i: (i, 0)),
            pl.BlockSpec((block_rows, d), lambda i: (i, 0)),
        ],
        out_specs=pl.BlockSpec((block_rows, d), lambda i: (i, 0)),
        out_shape=jax.ShapeDtypeStruct((n, d), jnp.float32),
    )(p0, p1)


def _segsum_body(n_pad, d, nh, hw_hbm, src_hbm, dst_hbm, zeros_hbm, out_hbm,
                 sidx, didx, rows, agg_sh, gsem, ssem):
    cid = lax.axis_index("c")
    sid = lax.axis_index("s")
    wid = sid * NC + cid
    stripe = n_pad // NS
    base = sid * stripe

    nr = NBLK_W // (2 * nh)
    tail = NBLK_W - nr * 2 * nh

    pltpu.sync_copy(zeros_hbm, agg_sh.at[pl.ds(base, stripe)])
    pltpu.sync_copy(src_hbm.at[wid], sidx)
    pltpu.sync_copy(dst_hbm.at[wid], didx)
    plsc.subcore_barrier()

    def gather(j, b):
        return pltpu.async_copy(hw_hbm.at[sidx.at[j]], rows.at[b], gsem)

    def scat(j, b):
        return pltpu.async_copy(rows.at[b], agg_sh.at[didx.at[j]], ssem,
                                add=True)

    def drain(sem, k):
        for _ in range(k):
            pltpu.make_async_copy(hw_hbm.at[pl.ds(0, EBLK)], rows.at[0],
                                  sem).wait()

    def round_body(r, _):
        j0 = r * 2 * nh

        @pl.when(r > 0)
        def _():
            drain(ssem, nh)
        gd = [gather(j0 + b, b) for b in range(nh)]
        for g in gd:
            g.wait()
        for b in range(nh):
            scat(j0 + b, b)

        @pl.when(r > 0)
        def _():
            drain(ssem, nh)
        gd = [gather(j0 + nh + b, nh + b) for b in range(nh)]
        for g in gd:
            g.wait()
        for b in range(nh):
            scat(j0 + nh + b, nh + b)
        return 0

    lax.fori_loop(0, nr, round_body, 0)
    drain(ssem, 2 * nh)
    for t in range(tail):
        j = nr * 2 * nh + t
        gather(j, t).wait()
        scat(j, t).wait()
    plsc.subcore_barrier()

    pltpu.sync_copy(agg_sh.at[pl.ds(base, stripe)],
                    out_hbm.at[cid, pl.ds(base, stripe)])


def _sc_segsum(hw, src3, dst3, zeros, n_pad):
    d = hw.shape[1]
    nh = 1 if d > 64 else 3
    mesh = plsc.VectorSubcoreMesh(core_axis_name="c", subcore_axis_name="s")
    kern = pl.kernel(
        functools.partial(_segsum_body, n_pad, d, nh),
        out_type=jax.ShapeDtypeStruct((NC, n_pad, d), jnp.float32),
        mesh=mesh,
        scratch_types=[
            pltpu.VMEM((NBLK_W, EBLK), jnp.int32),
            pltpu.VMEM((NBLK_W, EBLK), jnp.int32),
            pltpu.VMEM((2 * nh, EBLK, d), jnp.float32),
            pltpu.VMEM_SHARED((n_pad, d), jnp.float32),
            pltpu.SemaphoreType.DMA,
            pltpu.SemaphoreType.DMA,
        ],
        compiler_params=pltpu.CompilerParams(use_tc_tiling_on_sc=False),
    )
    return kern(hw, src3, dst3, zeros)


def _decoder_body(d, nblk_w, h2_hbm, aidx_hbm, bidx_hbm, out_hbm,
                  aidx, bidx, urows, vrows, outv, gsem):
    cid = lax.axis_index("c")
    sid = lax.axis_index("s")
    wid = sid * NC + cid

    pltpu.sync_copy(aidx_hbm.at[wid], aidx)
    pltpu.sync_copy(bidx_hbm.at[wid], bidx)

    def gathers(j, s):
        pltpu.async_copy(h2_hbm.at[aidx.at[j]], urows.at[s], gsem)
        pltpu.async_copy(h2_hbm.at[bidx.at[j]], vrows.at[s], gsem)

    def drain_pair():
        for _ in range(2):
            pltpu.make_async_copy(h2_hbm.at[pl.ds(0, PBLK)], urows.at[0],
                                  gsem).wait()

    def compute(j, s):
        def group(g, _):
            pv = g * L + lax.iota(jnp.int32, L)
            acc = jnp.zeros((L,), jnp.float32)
            for dd in range(d):
                dv = jnp.full((L,), dd, jnp.int32)
                acc = acc + (plsc.load_gather(urows.at[s], [pv, dv])
                             * plsc.load_gather(vrows.at[s], [pv, dv]))
            outv[s, pl.ds(g * L, L)] = acc
            return 0

        lax.fori_loop(0, PBLK // L, group, 0)
        pltpu.sync_copy(outv.at[s],
                        out_hbm.at[pl.ds((wid * nblk_w + j) * PBLK, PBLK)])

    gathers(0, 0)

    def round_body(r, _):
        j0 = 2 * r
        gathers(j0 + 1, 1)
        drain_pair()
        compute(j0, 0)

        @pl.when(j0 + 2 < nblk_w)
        def _():
            gathers(j0 + 2, 0)
        drain_pair()
        compute(j0 + 1, 1)
        return 0

    lax.fori_loop(0, nblk_w // 2, round_body, 0)


def _sc_decoder(h2, aidx3, bidx3):
    nblk_w = aidx3.shape[1]
    p = NW * nblk_w * PBLK
    d = h2.shape[1]
    mesh = plsc.VectorSubcoreMesh(core_axis_name="c", subcore_axis_name="s")
    kern = pl.kernel(
        functools.partial(_decoder_body, d, nblk_w),
        out_type=jax.ShapeDtypeStruct((p,), jnp.float32),
        mesh=mesh,
        scratch_types=[
            pltpu.VMEM((nblk_w, PBLK), jnp.int32),
            pltpu.VMEM((nblk_w, PBLK), jnp.int32),
            pltpu.VMEM((2, PBLK, d), jnp.float32),
            pltpu.VMEM((2, PBLK, d), jnp.float32),
            pltpu.VMEM((2, PBLK), jnp.float32),
            pltpu.SemaphoreType.DMA,
        ],
        compiler_params=pltpu.CompilerParams(
            use_tc_tiling_on_sc=False, needs_layout_passes=False),
    )
    return kern(h2, aidx3, bidx3)


def kernel(features, edge_index, positive_edge_pairs, negative_edge_pairs,
           W1, b1, W2, b2):
    src3 = edge_index[0].astype(jnp.int32).reshape(NW, NBLK_W, EBLK)
    dst3 = edge_index[1].astype(jnp.int32).reshape(NW, NBLK_W, EBLK)
    zeros128 = jnp.zeros((N_PAD // NS, 128), jnp.float32)
    zeros64 = jnp.zeros((N_PAD // NS, 64), jnp.float32)

    hw1 = _tc_matmul_bias(features, W1, b1)
    part1 = _sc_segsum(hw1, src3, dst3, zeros128, N_PAD)

    hw2 = _tc_relu_combine_matmul(part1[0], part1[1], W2, b2)
    part2 = _sc_segsum(hw2, src3, dst3, zeros64, N_PAD)
    h2 = _tc_add(part2[0], part2[1])

    all_pairs = jnp.concatenate(
        (positive_edge_pairs, negative_edge_pairs), axis=-1).astype(jnp.int32)
    npairs = all_pairs.shape[1]
    pad = (-npairs) % (NW * PBLK)
    nblk_w = (npairs + pad) // (NW * PBLK)
    aidx3 = jnp.pad(all_pairs[0], (0, pad)).reshape(NW, nblk_w, PBLK)
    bidx3 = jnp.pad(all_pairs[1], (0, pad)).reshape(NW, nblk_w, PBLK)
    out = _sc_decoder(h2, aidx3, bidx3)
    return out[:npairs]


# speedup vs baseline: 8.9455x; 1.0139x over previous
import functools
import jax
import jax.numpy as jnp
from jax import lax
from jax.experimental import pallas as pl
from jax.experimental.pallas import tpu as pltpu
from jax.experimental.pallas import tpu_sc as plsc

NC = 2
NS = 16
L = 16
NW = NC * NS

N_NODES = 10000
N_PAD = 10240
EBLK = 100
EPW = 10000
NBLK_W = EPW // EBLK
PBLK = 128


def _mm_bias_body(x_ref, w_ref, b_ref, o_ref):
    o_ref[...] = (
        jnp.dot(x_ref[...], w_ref[...], preferred_element_type=jnp.float32)
        + b_ref[...]
    )


def _tc_matmul_bias(x, w, b, block_rows=1000):
    n, d = x.shape
    h = w.shape[1]
    b2d = b.reshape(1, h)
    return pl.pallas_call(
        _mm_bias_body,
        grid=(n // block_rows,),
        in_specs=[
            pl.BlockSpec((block_rows, d), lambda i: (i, 0)),
            pl.BlockSpec((d, h), lambda i: (0, 0)),
            pl.BlockSpec((1, h), lambda i: (0, 0)),
        ],
        out_specs=pl.BlockSpec((block_rows, h), lambda i: (i, 0)),
        out_shape=jax.ShapeDtypeStruct((n, h), jnp.float32),
    )(x, w, b2d)


def _fuse_body(p0_ref, p1_ref, w_ref, b_ref, o_ref):
    h = jnp.maximum(p0_ref[...] + p1_ref[...], 0.0)
    o_ref[...] = (
        jnp.dot(h, w_ref[...], preferred_element_type=jnp.float32) + b_ref[...]
    )


def _tc_relu_combine_matmul(p0, p1, w, b, block_rows=1024):
    n, d = p0.shape
    h = w.shape[1]
    b2d = b.reshape(1, h)
    return pl.pallas_call(
        _fuse_body,
        grid=(n // block_rows,),
        in_specs=[
            pl.BlockSpec((block_rows, d), lambda i: (i, 0)),
            pl.BlockSpec((block_rows, d), lambda i: (i, 0)),
            pl.BlockSpec((d, h), lambda i: (0, 0)),
            pl.BlockSpec((1, h), lambda i: (0, 0)),
        ],
        out_specs=pl.BlockSpec((block_rows, h), lambda i: (i, 0)),
        out_shape=jax.ShapeDtypeStruct((n, h), jnp.float32),
    )(p0, p1, w, b2d)


def _add_body(p0_ref, p1_ref, o_ref):
    o_ref[...] = p0_ref[...] + p1_ref[...]


def _tc_add(p0, p1, block_rows=2048):
    n, d = p0.shape
    return pl.pallas_call(
        _add_body,
        grid=(n // block_rows,),
        in_specs=[
            pl.BlockSpec((block_rows, d), lambda i: (i, 0)),
            pl.BlockSpec((block_rows, d), lambda i: (i, 0)),
        ],
        out_specs=pl.BlockSpec((block_rows, d), lambda i: (i, 0)),
        out_shape=jax.ShapeDtypeStruct((n, d), jnp.float32),
    )(p0, p1)


def _segsum_body(n_pad, d, nh, hw_hbm, src_hbm, dst_hbm, zeros_hbm, out_hbm,
                 sidx, didx, rows, agg_sh, gsem, ssemA, ssemB):
    cid = lax.axis_index("c")
    sid = lax.axis_index("s")
    wid = sid * NC + cid
    stripe = n_pad // NS
    base = sid * stripe

    nr = NBLK_W // (2 * nh)
    tail = NBLK_W - nr * 2 * nh

    pltpu.sync_copy(zeros_hbm, agg_sh.at[pl.ds(base, stripe)])
    pltpu.sync_copy(src_hbm.at[wid], sidx)
    pltpu.sync_copy(dst_hbm.at[wid], didx)
    plsc.subcore_barrier()

    def gather(j, b):
        return pltpu.async_copy(hw_hbm.at[sidx.at[j]], rows.at[b], gsem)

    def scat(j, b, sem):
        return pltpu.async_copy(rows.at[b], agg_sh.at[didx.at[j]], sem,
                                add=True)

    def drain(sem, k):
        for _ in range(k):
            pltpu.make_async_copy(hw_hbm.at[pl.ds(0, EBLK)], rows.at[0],
                                  sem).wait()

    def round_body(r, _):
        j0 = r * 2 * nh

        @pl.when(r > 0)
        def _():
            drain(ssemA, nh)
        gd = [gather(j0 + b, b) for b in range(nh)]
        for g in gd:
            g.wait()
        for b in range(nh):
            scat(j0 + b, b, ssemA)

        @pl.when(r > 0)
        def _():
            drain(ssemB, nh)
        gd = [gather(j0 + nh + b, nh + b) for b in range(nh)]
        for g in gd:
            g.wait()
        for b in range(nh):
            scat(j0 + nh + b, nh + b, ssemB)
        return 0

    lax.fori_loop(0, nr, round_body, 0)
    drain(ssemA, nh)
    drain(ssemB, nh)
    for t in range(tail):
        j = nr * 2 * nh + t
        gather(j, t).wait()
        scat(j, t, ssemA).wait()
    plsc.subcore_barrier()

    pltpu.sync_copy(agg_sh.at[pl.ds(base, stripe)],
                    out_hbm.at[cid, pl.ds(base, stripe)])


def _sc_segsum(hw, src3, dst3, zeros, n_pad):
    d = hw.shape[1]
    nh = 1 if d > 64 else 5
    mesh = plsc.VectorSubcoreMesh(core_axis_name="c", subcore_axis_name="s")
    kern = pl.kernel(
        functools.partial(_segsum_body, n_pad, d, nh),
        out_type=jax.ShapeDtypeStruct((NC, n_pad, d), jnp.float32),
        mesh=mesh,
        scratch_types=[
            pltpu.VMEM((NBLK_W, EBLK), jnp.int32),
            pltpu.VMEM((NBLK_W, EBLK), jnp.int32),
            pltpu.VMEM((2 * nh, EBLK, d), jnp.float32),
            pltpu.VMEM_SHARED((n_pad, d), jnp.float32),
            pltpu.SemaphoreType.DMA,
            pltpu.SemaphoreType.DMA,
            pltpu.SemaphoreType.DMA,
        ],
        compiler_params=pltpu.CompilerParams(use_tc_tiling_on_sc=False),
    )
    return kern(hw, src3, dst3, zeros)


def _decoder_body(d, nblk_w, h2_hbm, aidx_hbm, bidx_hbm, out_hbm,
                  aidx, bidx, urows, vrows, outv, gsem0, gsem1):
    cid = lax.axis_index("c")
    sid = lax.axis_index("s")
    wid = sid * NC + cid

    pltpu.sync_copy(aidx_hbm.at[wid], aidx)
    pltpu.sync_copy(bidx_hbm.at[wid], bidx)

    def gathers(j, s):
        sem = gsem0 if s == 0 else gsem1
        pltpu.async_copy(h2_hbm.at[aidx.at[j]], urows.at[s], sem)
        pltpu.async_copy(h2_hbm.at[bidx.at[j]], vrows.at[s], sem)

    def drain_pair(s):
        sem = gsem0 if s == 0 else gsem1
        for _ in range(2):
            pltpu.make_async_copy(h2_hbm.at[pl.ds(0, PBLK)], urows.at[0],
                                  sem).wait()

    def compute(j, s):
        def group(g, _):
            pv = g * L + lax.iota(jnp.int32, L)
            accs = [jnp.zeros((L,), jnp.float32) for _ in range(4)]
            for dd in range(d):
                dv = jnp.full((L,), dd, jnp.int32)
                accs[dd % 4] = accs[dd % 4] + (
                    plsc.load_gather(urows.at[s], [pv, dv])
                    * plsc.load_gather(vrows.at[s], [pv, dv]))
            outv[s, pl.ds(g * L, L)] = (
                (accs[0] + accs[1]) + (accs[2] + accs[3]))
            return 0

        lax.fori_loop(0, PBLK // L, group, 0)
        pltpu.sync_copy(outv.at[s],
                        out_hbm.at[pl.ds((wid * nblk_w + j) * PBLK, PBLK)])

    gathers(0, 0)

    def round_body(r, _):
        j0 = 2 * r
        gathers(j0 + 1, 1)
        drain_pair(0)
        compute(j0, 0)

        @pl.when(j0 + 2 < nblk_w)
        def _():
            gathers(j0 + 2, 0)
        drain_pair(1)
        compute(j0 + 1, 1)
        return 0

    lax.fori_loop(0, nblk_w // 2, round_body, 0)


def _sc_decoder(h2, aidx3, bidx3):
    nblk_w = aidx3.shape[1]
    p = NW * nblk_w * PBLK
    d = h2.shape[1]
    mesh = plsc.VectorSubcoreMesh(core_axis_name="c", subcore_axis_name="s")
    kern = pl.kernel(
        functools.partial(_decoder_body, d, nblk_w),
        out_type=jax.ShapeDtypeStruct((p,), jnp.float32),
        mesh=mesh,
        scratch_types=[
            pltpu.VMEM((nblk_w, PBLK), jnp.int32),
            pltpu.VMEM((nblk_w, PBLK), jnp.int32),
            pltpu.VMEM((2, PBLK, d), jnp.float32),
            pltpu.VMEM((2, PBLK, d), jnp.float32),
            pltpu.VMEM((2, PBLK), jnp.float32),
            pltpu.SemaphoreType.DMA,
            pltpu.SemaphoreType.DMA,
        ],
        compiler_params=pltpu.CompilerParams(
            use_tc_tiling_on_sc=False, needs_layout_passes=False),
    )
    return kern(h2, aidx3, bidx3)


def kernel(features, edge_index, positive_edge_pairs, negative_edge_pairs,
           W1, b1, W2, b2):
    src3 = edge_index[0].astype(jnp.int32).reshape(NW, NBLK_W, EBLK)
    dst3 = edge_index[1].astype(jnp.int32).reshape(NW, NBLK_W, EBLK)
    zeros128 = jnp.zeros((N_PAD // NS, 128), jnp.float32)
    zeros64 = jnp.zeros((N_PAD // NS, 64), jnp.float32)

    hw1 = _tc_matmul_bias(features, W1, b1)
    part1 = _sc_segsum(hw1, src3, dst3, zeros128, N_PAD)

    hw2 = _tc_relu_combine_matmul(part1[0], part1[1], W2, b2)
    part2 = _sc_segsum(hw2, src3, dst3, zeros64, N_PAD)
    h2 = _tc_add(part2[0], part2[1])

    all_pairs = jnp.concatenate(
        (positive_edge_pairs, negative_edge_pairs), axis=-1).astype(jnp.int32)
    npairs = all_pairs.shape[1]
    pad = (-npairs) % (NW * PBLK)
    nblk_w = (npairs + pad) // (NW * PBLK)
    aidx3 = jnp.pad(all_pairs[0], (0, pad)).reshape(NW, nblk_w, PBLK)
    bidx3 = jnp.pad(all_pairs[1], (0, pad)).reshape(NW, nblk_w, PBLK)
    out = _sc_decoder(h2, aidx3, bidx3)
    return out[:npairs]


# speedup vs baseline: 9.2800x; 1.0374x over previous
import functools
import jax
import jax.numpy as jnp
from jax import lax
from jax.experimental import pallas as pl
from jax.experimental.pallas import tpu as pltpu
from jax.experimental.pallas import tpu_sc as plsc

NC = 2
NS = 16
L = 16
NW = NC * NS

N_NODES = 10000
N_PAD = 10240
EBLK = 100
EPW = 10000
NBLK_W = EPW // EBLK
CS_NBLK = 200
PBLK = 128


def _mm_bias_split_body(x_ref, w_ref, b_ref, o_ref):
    r = (jnp.dot(x_ref[...], w_ref[...], preferred_element_type=jnp.float32)
         + b_ref[...])
    h = r.shape[-1] // 2
    o_ref[0, :, :] = r[:, :h]
    o_ref[1, :, :] = r[:, h:]


def _tc_matmul_bias_split(x, w, b, block_rows=1000):
    n, d = x.shape
    h = w.shape[1]
    b2d = b.reshape(1, h)
    return pl.pallas_call(
        _mm_bias_split_body,
        grid=(n // block_rows,),
        in_specs=[
            pl.BlockSpec((block_rows, d), lambda i: (i, 0)),
            pl.BlockSpec((d, h), lambda i: (0, 0)),
            pl.BlockSpec((1, h), lambda i: (0, 0)),
        ],
        out_specs=pl.BlockSpec((2, block_rows, h // 2), lambda i: (0, i, 0)),
        out_shape=jax.ShapeDtypeStruct((2, n, h // 2), jnp.float32),
    )(x, w, b2d)


def _fuse_cs_body(p0_ref, p1_ref, w_ref, b_ref, o_ref):
    hh = p0_ref.shape[-1]
    h0 = jnp.maximum(p0_ref[...], 0.0)
    h1 = jnp.maximum(p1_ref[...], 0.0)
    o_ref[...] = (
        jnp.dot(h0, w_ref[0:hh, :], preferred_element_type=jnp.float32)
        + jnp.dot(h1, w_ref[hh:, :], preferred_element_type=jnp.float32)
        + b_ref[...]
    )


def _tc_relu_combine_matmul(p0, p1, w, b, block_rows=1024):
    n, d = p0.shape
    h = w.shape[1]
    b2d = b.reshape(1, h)
    return pl.pallas_call(
        _fuse_cs_body,
        grid=(n // block_rows,),
        in_specs=[
            pl.BlockSpec((block_rows, d), lambda i: (i, 0)),
            pl.BlockSpec((block_rows, d), lambda i: (i, 0)),
            pl.BlockSpec((2 * d, h), lambda i: (0, 0)),
            pl.BlockSpec((1, h), lambda i: (0, 0)),
        ],
        out_specs=pl.BlockSpec((block_rows, h), lambda i: (i, 0)),
        out_shape=jax.ShapeDtypeStruct((n, h), jnp.float32),
    )(p0, p1, w, b2d)


def _add_body(p0_ref, p1_ref, o_ref):
    o_ref[...] = p0_ref[...] + p1_ref[...]


def _tc_add(p0, p1, block_rows=2048):
    n, d = p0.shape
    return pl.pallas_call(
        _add_body,
        grid=(n // block_rows,),
        in_specs=[
            pl.BlockSpec((block_rows, d), lambda i: (i, 0)),
            pl.BlockSpec((block_rows, d), lambda i: (i, 0)),
        ],
        out_specs=pl.BlockSpec((block_rows, d), lambda i: (i, 0)),
        out_shape=jax.ShapeDtypeStruct((n, d), jnp.float32),
    )(p0, p1)


def _segsum_body(n_pad, d, nh, hw_hbm, src_hbm, dst_hbm, zeros_hbm, out_hbm,
                 sidx, didx, rows, agg_sh, gsem, ssemA, ssemB):
    cid = lax.axis_index("c")
    sid = lax.axis_index("s")
    wid = sid * NC + cid
    stripe = n_pad // NS
    base = sid * stripe

    nr = NBLK_W // (2 * nh)
    tail = NBLK_W - nr * 2 * nh

    pltpu.sync_copy(zeros_hbm, agg_sh.at[pl.ds(base, stripe)])
    pltpu.sync_copy(src_hbm.at[wid], sidx)
    pltpu.sync_copy(dst_hbm.at[wid], didx)
    plsc.subcore_barrier()

    def gather(j, b):
        return pltpu.async_copy(hw_hbm.at[sidx.at[j]], rows.at[b], gsem)

    def scat(j, b, sem):
        return pltpu.async_copy(rows.at[b], agg_sh.at[didx.at[j]], sem,
                                add=True)

    def drain(sem, k):
        for _ in range(k):
            pltpu.make_async_copy(hw_hbm.at[pl.ds(0, EBLK)], rows.at[0],
                                  sem).wait()

    def round_body(r, _):
        j0 = r * 2 * nh

        @pl.when(r > 0)
        def _():
            drain(ssemA, nh)
        gd = [gather(j0 + b, b) for b in range(nh)]
        for g in gd:
            g.wait()
        for b in range(nh):
            scat(j0 + b, b, ssemA)

        @pl.when(r > 0)
        def _():
            drain(ssemB, nh)
        gd = [gather(j0 + nh + b, nh + b) for b in range(nh)]
        for g in gd:
            g.wait()
        for b in range(nh):
            scat(j0 + nh + b, nh + b, ssemB)
        return 0

    lax.fori_loop(0, nr, round_body, 0)
    drain(ssemA, nh)
    drain(ssemB, nh)
    for t in range(tail):
        j = nr * 2 * nh + t
        gather(j, t).wait()
        scat(j, t, ssemA).wait()
    plsc.subcore_barrier()

    pltpu.sync_copy(agg_sh.at[pl.ds(base, stripe)],
                    out_hbm.at[cid, pl.ds(base, stripe)])


def _sc_segsum(hw, src3, dst3, zeros, n_pad):
    d = hw.shape[1]
    nh = 1 if d > 64 else 5
    mesh = plsc.VectorSubcoreMesh(core_axis_name="c", subcore_axis_name="s")
    kern = pl.kernel(
        functools.partial(_segsum_body, n_pad, d, nh),
        out_type=jax.ShapeDtypeStruct((NC, n_pad, d), jnp.float32),
        mesh=mesh,
        scratch_types=[
            pltpu.VMEM((NBLK_W, EBLK), jnp.int32),
            pltpu.VMEM((NBLK_W, EBLK), jnp.int32),
            pltpu.VMEM((2 * nh, EBLK, d), jnp.float32),
            pltpu.VMEM_SHARED((n_pad, d), jnp.float32),
            pltpu.SemaphoreType.DMA,
            pltpu.SemaphoreType.DMA,
            pltpu.SemaphoreType.DMA,
        ],
        compiler_params=pltpu.CompilerParams(use_tc_tiling_on_sc=False),
    )
    return kern(hw, src3, dst3, zeros)


def _segsum_cs_body(n_pad, d, nh, hw_hbm, src_hbm, dst_hbm, zeros_hbm, out_hbm,
                    sidx, didx, rows, agg_sh, gsem, ssemA, ssemB):
    cid = lax.axis_index("c")
    sid = lax.axis_index("s")
    stripe = n_pad // NS
    base = sid * stripe

    nr = CS_NBLK // (2 * nh)
    tail = CS_NBLK - nr * 2 * nh

    pltpu.sync_copy(zeros_hbm, agg_sh.at[pl.ds(base, stripe)])
    pltpu.sync_copy(src_hbm.at[cid, sid], sidx)
    pltpu.sync_copy(dst_hbm.at[sid], didx)
    plsc.subcore_barrier()

    def gather(j, b):
        return pltpu.async_copy(hw_hbm.at[sidx.at[j]], rows.at[b], gsem)

    def scat(j, b, sem):
        return pltpu.async_copy(rows.at[b], agg_sh.at[didx.at[j]], sem,
                                add=True)

    def drain(sem, k):
        for _ in range(k):
            pltpu.make_async_copy(hw_hbm.at[pl.ds(0, EBLK)], rows.at[0],
                                  sem).wait()

    def round_body(r, _):
        j0 = r * 2 * nh

        @pl.when(r > 0)
        def _():
            drain(ssemA, nh)
        gd = [gather(j0 + b, b) for b in range(nh)]
        for g in gd:
            g.wait()
        for b in range(nh):
            scat(j0 + b, b, ssemA)

        @pl.when(r > 0)
        def _():
            drain(ssemB, nh)
        gd = [gather(j0 + nh + b, nh + b) for b in range(nh)]
        for g in gd:
            g.wait()
        for b in range(nh):
            scat(j0 + nh + b, nh + b, ssemB)
        return 0

    lax.fori_loop(0, nr, round_body, 0)
    drain(ssemA, nh)
    drain(ssemB, nh)
    for t in range(tail):
        j = nr * 2 * nh + t
        gather(j, t).wait()
        scat(j, t, ssemA).wait()
    plsc.subcore_barrier()

    pltpu.sync_copy(agg_sh.at[pl.ds(base, stripe)],
                    out_hbm.at[cid, pl.ds(base, stripe)])


def _sc_segsum_colsplit(hwflat, src4, dst16, zeros, n_pad):
    d = hwflat.shape[1]
    nh = 3
    mesh = plsc.VectorSubcoreMesh(core_axis_name="c", subcore_axis_name="s")
    kern = pl.kernel(
        functools.partial(_segsum_cs_body, n_pad, d, nh),
        out_type=jax.ShapeDtypeStruct((NC, n_pad, d), jnp.float32),
        mesh=mesh,
        scratch_types=[
            pltpu.VMEM((CS_NBLK, EBLK), jnp.int32),
            pltpu.VMEM((CS_NBLK, EBLK), jnp.int32),
            pltpu.VMEM((2 * nh, EBLK, d), jnp.float32),
            pltpu.VMEM_SHARED((n_pad, d), jnp.float32),
            pltpu.SemaphoreType.DMA,
            pltpu.SemaphoreType.DMA,
            pltpu.SemaphoreType.DMA,
        ],
        compiler_params=pltpu.CompilerParams(use_tc_tiling_on_sc=False),
    )
    return kern(hwflat, src4, dst16, zeros)


def _decoder_body(d, nblk_w, h2_hbm, aidx_hbm, bidx_hbm, out_hbm,
                  aidx, bidx, urows, vrows, outv, gsem0, gsem1):
    cid = lax.axis_index("c")
    sid = lax.axis_index("s")
    wid = sid * NC + cid

    pltpu.sync_copy(aidx_hbm.at[wid], aidx)
    pltpu.sync_copy(bidx_hbm.at[wid], bidx)

    def gathers(j, s):
        sem = gsem0 if s == 0 else gsem1
        pltpu.async_copy(h2_hbm.at[aidx.at[j]], urows.at[s], sem)
        pltpu.async_copy(h2_hbm.at[bidx.at[j]], vrows.at[s], sem)

    def drain_pair(s):
        sem = gsem0 if s == 0 else gsem1
        for _ in range(2):
            pltpu.make_async_copy(h2_hbm.at[pl.ds(0, PBLK)], urows.at[0],
                                  sem).wait()

    def compute(j, s):
        def group(g, _):
            iot = lax.iota(jnp.int32, L)
            pv = g * L + iot
            accs = [jnp.zeros((L,), jnp.float32) for _ in range(4)]
            for dd in range(d):
                dvr = jnp.bitwise_and(iot + dd, d - 1)
                accs[dd % 4] = accs[dd % 4] + (
                    plsc.load_gather(urows.at[s], [pv, dvr])
                    * plsc.load_gather(vrows.at[s], [pv, dvr]))
            outv[s, pl.ds(g * L, L)] = (
                (accs[0] + accs[1]) + (accs[2] + accs[3]))
            return 0

        lax.fori_loop(0, PBLK // L, group, 0)
        pltpu.sync_copy(outv.at[s],
                        out_hbm.at[pl.ds((wid * nblk_w + j) * PBLK, PBLK)])

    gathers(0, 0)

    def round_body(r, _):
        j0 = 2 * r
        gathers(j0 + 1, 1)
        drain_pair(0)
        compute(j0, 0)

        @pl.when(j0 + 2 < nblk_w)
        def _():
            gathers(j0 + 2, 0)
        drain_pair(1)
        compute(j0 + 1, 1)
        return 0

    lax.fori_loop(0, nblk_w // 2, round_body, 0)


def _sc_decoder(h2, aidx3, bidx3):
    nblk_w = aidx3.shape[1]
    p = NW * nblk_w * PBLK
    d = h2.shape[1]
    mesh = plsc.VectorSubcoreMesh(core_axis_name="c", subcore_axis_name="s")
    kern = pl.kernel(
        functools.partial(_decoder_body, d, nblk_w),
        out_type=jax.ShapeDtypeStruct((p,), jnp.float32),
        mesh=mesh,
        scratch_types=[
            pltpu.VMEM((nblk_w, PBLK), jnp.int32),
            pltpu.VMEM((nblk_w, PBLK), jnp.int32),
            pltpu.VMEM((2, PBLK, d), jnp.float32),
            pltpu.VMEM((2, PBLK, d), jnp.float32),
            pltpu.VMEM((2, PBLK), jnp.float32),
            pltpu.SemaphoreType.DMA,
            pltpu.SemaphoreType.DMA,
        ],
        compiler_params=pltpu.CompilerParams(
            use_tc_tiling_on_sc=False, needs_layout_passes=False),
    )
    return kern(h2, aidx3, bidx3)


def kernel(features, edge_index, positive_edge_pairs, negative_edge_pairs,
           W1, b1, W2, b2):
    src = edge_index[0].astype(jnp.int32)
    dst = edge_index[1].astype(jnp.int32)
    src3 = src.reshape(NW, NBLK_W, EBLK)
    dst3 = dst.reshape(NW, NBLK_W, EBLK)
    src16 = src.reshape(NS, CS_NBLK, EBLK)
    src4 = jnp.stack([src16, src16 + N_NODES])
    dst16 = dst.reshape(NS, CS_NBLK, EBLK)
    zeros64 = jnp.zeros((N_PAD // NS, 64), jnp.float32)

    hw1s = _tc_matmul_bias_split(features, W1, b1)
    part1 = _sc_segsum_colsplit(hw1s.reshape(2 * N_NODES, 64), src4, dst16,
                                zeros64, N_PAD)

    hw2 = _tc_relu_combine_matmul(part1[0], part1[1], W2, b2)
    part2 = _sc_segsum(hw2, src3, dst3, zeros64, N_PAD)
    h2 = _tc_add(part2[0], part2[1])

    all_pairs = jnp.concatenate(
        (positive_edge_pairs, negative_edge_pairs), axis=-1).astype(jnp.int32)
    npairs = all_pairs.shape[1]
    pad = (-npairs) % (NW * PBLK)
    nblk_w = (npairs + pad) // (NW * PBLK)
    aidx3 = jnp.pad(all_pairs[0], (0, pad)).reshape(NW, nblk_w, PBLK)
    bidx3 = jnp.pad(all_pairs[1], (0, pad)).reshape(NW, nblk_w, PBLK)
    out = _sc_decoder(h2, aidx3, bidx3)
    return out[:npairs]
